# baseline (speedup 1.0000x reference)
import jax
import jax.numpy as jnp
from jax.experimental import pallas as pl

_NUM_CLASSES = 1000
_LS = 0.1
_COLD = _LS / (_NUM_CLASSES - 1)
_HOT = (1.0 - _LS) + _COLD

_S_BLK = 5


def _onehot_body(xt_ref, o_ref):
    _, s, b = xt_ref.shape
    iota = jax.lax.broadcasted_iota(jnp.int32, (s, _NUM_CLASSES, b), 1)
    o_ref[...] = jnp.where(
        xt_ref[0][:, None, :] == iota,
        jnp.float32(_HOT),
        jnp.float32(_COLD),
    )


def kernel(x_i):
    b, s = x_i.shape
    xt = x_i.T.reshape(s // _S_BLK, _S_BLK, b)
    out_t = pl.pallas_call(
        _onehot_body,
        grid=(s // _S_BLK,),
        in_specs=[pl.BlockSpec((1, _S_BLK, b), lambda i: (i, 0, 0))],
        out_specs=pl.BlockSpec((_S_BLK, _NUM_CLASSES, b), lambda i: (i, 0, 0)),
        out_shape=jax.ShapeDtypeStruct((s, _NUM_CLASSES, b), jnp.float32),
    )(xt)
    return out_t.transpose(2, 0, 1)


# speedup vs baseline: 1.0581x; 1.0581x over previous
import jax
import jax.numpy as jnp
from jax.experimental import pallas as pl

_NUM_CLASSES = 1000
_LS = 0.1
_COLD = _LS / (_NUM_CLASSES - 1)
_HOT = (1.0 - _LS) + _COLD

_C_BLK = 200
_B_BLK = 128


def _onehot_body(xt_ref, o_ref):
    ci = pl.program_id(0) * _C_BLK
    s, b = xt_ref.shape
    iota = ci + jax.lax.broadcasted_iota(jnp.int32, (s, _C_BLK, b), 1)
    o_ref[...] = jnp.where(
        xt_ref[...][:, None, :] == iota,
        jnp.float32(_HOT),
        jnp.float32(_COLD),
    )


def kernel(x_i):
    b, s = x_i.shape
    xt = x_i.T
    out_t = pl.pallas_call(
        _onehot_body,
        grid=(_NUM_CLASSES // _C_BLK, b // _B_BLK),
        in_specs=[pl.BlockSpec((s, _B_BLK), lambda i, j: (0, j))],
        out_specs=pl.BlockSpec((s, _C_BLK, _B_BLK), lambda i, j: (0, i, j)),
        out_shape=jax.ShapeDtypeStruct((s, _NUM_CLASSES, b), jnp.float32),
    )(xt)
    return out_t.transpose(2, 0, 1)
